# baseline (speedup 1.0000x reference)
import functools

import jax
import jax.numpy as jnp
from jax import lax
from jax.experimental import pallas as pl
from jax.experimental.pallas import tpu as pltpu
from jax.experimental.pallas import tpu_sc as plsc

_F32 = jnp.float32
_BF16 = jnp.bfloat16
_NC = 2
_NS = 16
_HI = jax.lax.Precision.HIGHEST


def _lr(x):
    return jnp.where(x >= 0, x, 0.2 * x)


def _dot(a, b):
    return jnp.dot(a.astype(_BF16) if a.dtype == _F32 else a,
                   b.astype(_BF16) if b.dtype == _F32 else b,
                   preferred_element_type=_F32)


@functools.lru_cache(maxsize=None)
def _make_gather(R, C, K, P=8):
    NW = _NC * _NS
    rows_per_w = R // NW
    n_chunks = rows_per_w // P
    mesh = plsc.VectorSubcoreMesh(core_axis_name="c", subcore_axis_name="s")

    @functools.partial(
        pl.kernel,
        mesh=mesh,
        out_type=jax.ShapeDtypeStruct((R * K, C), _F32),
        scratch_types=[
            pltpu.VMEM((P * K,), jnp.int32),
            pltpu.VMEM((P * K, C), _F32),
            pltpu.SemaphoreType.DMA,
        ],
        compiler_params=pltpu.CompilerParams(use_tc_tiling_on_sc=False),
    )
    def gk(x_hbm, idx_hbm, out_hbm, idx_v, rows_v, sem):
        wid = lax.axis_index("s") * _NC + lax.axis_index("c")
        base = wid * rows_per_w

        def body(g, carry):
            e0 = (base + g * P) * K
            pltpu.sync_copy(idx_hbm.at[pl.ds(e0, P * K)], idx_v)
            pltpu.async_copy(x_hbm.at[idx_v], rows_v, sem).wait()
            pltpu.sync_copy(rows_v, out_hbm.at[pl.ds(e0, P * K)])
            return carry

        lax.fori_loop(0, n_chunks, body, 0)

    return gk


def _edge_body(K, TR, cin, x_ref, g_ref, w_ref, b_ref, o_ref):
    x = x_ref[:, :cin]
    g = g_ref[...].reshape(TR, K, -1)[:, :, :cin]
    d16 = (g - x[:, None, :]).astype(_BF16)
    c16 = jnp.broadcast_to(x.astype(_BF16)[:, None, :], (TR, K, cin))
    e = jnp.concatenate([d16, c16], axis=2).reshape(TR * K, 2 * cin)
    h = _dot(e, w_ref[...]).reshape(TR, K, -1) + b_ref[...][None]
    o_ref[...] = jnp.max(_lr(h), axis=1)


def _edge_mm_max(X, G, W16, b2d, K, cin, TR=256):
    R, Cinp = X.shape
    Cout = W16.shape[1]
    grid = R // TR
    return pl.pallas_call(
        functools.partial(_edge_body, K, TR, cin),
        grid=(grid,),
        in_specs=[
            pl.BlockSpec((TR, Cinp), lambda i: (i, 0)),
            pl.BlockSpec((TR * K, Cinp), lambda i: (i, 0)),
            pl.BlockSpec((2 * cin, Cout), lambda i: (0, 0)),
            pl.BlockSpec((1, Cout), lambda i: (0, 0)),
        ],
        out_specs=pl.BlockSpec((TR, Cout), lambda i: (i, 0)),
        out_shape=jax.ShapeDtypeStruct((R, Cout), _F32),
    )(X, G, W16, b2d)


def _geo_layer(Xp, idxflat, W, b, cin, R, K):
    cinp = Xp.shape[1]
    w16 = W.astype(_BF16)
    G = _make_gather(R, cinp, K)(Xp, idxflat)
    return _edge_mm_max(Xp, G, w16, b[None, :], K, cin)


def _a_body(c1, x1_ref, x2_ref, x3_ref, wm_ref, rs_ref, jf_ref, o_ref):
    w = wm_ref[0]
    a = jnp.concatenate(
        [
            _dot(w, x1_ref[:, :c1]),
            _dot(w, x2_ref[...]),
            _dot(w, x3_ref[...]),
        ],
        axis=1,
    )
    o_ref[0] = jnp.concatenate([a / rs_ref[0], jf_ref[0]], axis=1)


def _a_call(X1, X2, X3, Wm, Rs, Jf, C1):
    BB, J, N = Wm.shape
    C1p, C2, C3 = X1.shape[1], X2.shape[1], X3.shape[1]
    CJ = Jf.shape[2]
    Cout = C1 + C2 + C3 + CJ
    return pl.pallas_call(
        functools.partial(_a_body, C1),
        grid=(BB,),
        in_specs=[
            pl.BlockSpec((N, C1p), lambda i: (i, 0)),
            pl.BlockSpec((N, C2), lambda i: (i, 0)),
            pl.BlockSpec((N, C3), lambda i: (i, 0)),
            pl.BlockSpec((1, J, N), lambda i: (i, 0, 0)),
            pl.BlockSpec((1, J, 1), lambda i: (i, 0, 0)),
            pl.BlockSpec((1, J, CJ), lambda i: (i, 0, 0)),
        ],
        out_specs=pl.BlockSpec((1, J, Cout), lambda i: (i, 0, 0)),
        out_shape=jax.ShapeDtypeStruct((BB, J, Cout), _F32),
    )(X1, X2, X3, Wm, Rs, Jf)


def _onehots(ski, J, KJ):
    R2 = ski.shape[0]
    col = lax.broadcasted_iota(jnp.int32, (R2, R2), 1)
    row = lax.broadcasted_iota(jnp.int32, (R2, R2), 0)
    blk = (col // J) == (row // J)
    colm = col % J
    return [jnp.where(blk & (colm == ski[:, k:k + 1]), 1.0, 0.0).astype(_F32)
            for k in range(KJ)]


def _sk_layer(x, ohs, w16, b):
    x16 = x.astype(_BF16)
    m = None
    for oh in ohs:
        nb = jnp.dot(oh, x, preferred_element_type=_F32, precision=_HI)
        d16 = (nb - x).astype(_BF16)
        e = jnp.concatenate([d16, x16], axis=1)
        h = _lr(_dot(e, w16) + b)
        m = h if m is None else jnp.maximum(m, h)
    return m


def _sk_triple(x, ohs, w):
    x1 = _sk_layer(x, ohs, w[0][0], w[0][1])
    x2 = _sk_layer(x1, ohs, w[1][0], w[1][1])
    x3 = _sk_layer(x2, ohs, w[2][0], w[2][1])
    return jnp.concatenate([x1, x2, x3], axis=1)


def _skc_body(J, KJ, cat_ref, ski_ref, w1, b1, w2, b2, w3, b3, o_ref):
    ohs = _onehots(ski_ref[...], J, KJ)
    w = ((w1[...], b1[...]), (w2[...], b2[...]), (w3[...], b3[...]))
    o_ref[...] = _sk_triple(cat_ref[...], ohs, w)


def _skc_call(catrows, ski_rows, wlist, J, KJ, Cout):
    R2 = catrows.shape[0]
    args = [catrows, ski_rows]
    for (w16, b2d) in wlist:
        args += [w16, b2d]
    return pl.pallas_call(
        functools.partial(_skc_body, J, KJ),
        out_shape=jax.ShapeDtypeStruct((R2, Cout), _F32),
    )(*args)


def _joint_body(sf_ref, rf_ref, w1, b1, w2, b2, w3, b3, o_ref):
    x = jnp.concatenate([sf_ref[...], rf_ref[...]], axis=1)
    x = _lr(_dot(x, w1[...]) + b1[...])
    x = _lr(_dot(x, w2[...]) + b2[...])
    o_ref[...] = _dot(x, w3[...]) + b3[...]


def _joint_call(sF, rF, w1, b1, w2, b2, w3, b3):
    R2 = sF.shape[0]
    Cout = w3.shape[1]
    return pl.pallas_call(
        _joint_body,
        out_shape=jax.ShapeDtypeStruct((R2, Cout), _F32),
    )(sF, rF, w1, b1, w2, b2, w3, b3)


def _res_body(J, KJ, x_ref, ski_ref,
              sw1, sb1, sw2, sb2, sw3, sb3,
              w1, b1, w2, b2, w3, b3, o_ref):
    i = pl.program_id(0)
    x = jnp.where(i == 0, x_ref[...], o_ref[...])
    ohs = _onehots(ski_ref[...], J, KJ)
    w = ((sw1[0], sb1[0]), (sw2[0], sb2[0]), (sw3[0], sb3[0]))
    f = _sk_triple(x, ohs, w)
    f = _lr(_dot(f, w1[0]) + b1[0])
    f = _lr(_dot(f, w2[0]) + b2[0])
    f = _dot(f, w3[0]) + b3[0]
    o_ref[...] = x + f


def _res_call(x, ski_rows, stacked, J, KJ):
    R2, C = x.shape
    nblk = stacked[0].shape[0]
    full = lambda a: pl.BlockSpec(a.shape, lambda i: tuple(0 for _ in a.shape))
    perblk = lambda a: pl.BlockSpec((1,) + a.shape[1:],
                                    lambda i: (i,) + tuple(0 for _ in a.shape[1:]))
    return pl.pallas_call(
        functools.partial(_res_body, J, KJ),
        grid=(nblk,),
        in_specs=[full(x), full(ski_rows)] + [perblk(a) for a in stacked],
        out_specs=pl.BlockSpec((R2, C), lambda i: (0, 0)),
        out_shape=jax.ShapeDtypeStruct((R2, C), _F32),
    )(x, ski_rows, *stacked)


def _last_body(J, KJ, x_ref, ski_ref,
               sw1, sb1, sw2, sb2, sw3, sb3,
               w1, b1, w2, b2, w3, b3, o_ref):
    ohs = _onehots(ski_ref[...], J, KJ)
    w = ((sw1[...], sb1[...]), (sw2[...], sb2[...]), (sw3[...], sb3[...]))
    f = _sk_triple(x_ref[...], ohs, w)
    f = _lr(_dot(f, w1[...]) + b1[...])
    f = _lr(_dot(f, w2[...]) + b2[...])
    o_ref[...] = _dot(f, w3[...]) + b3[...]


def _last_call(x, ski_rows, flat, Cout, J, KJ):
    R2 = x.shape[0]
    args = [x, ski_rows] + flat
    return pl.pallas_call(
        functools.partial(_last_body, J, KJ),
        out_shape=jax.ShapeDtypeStruct((R2, Cout), _F32),
    )(*args)


def _w16(W):
    return W.astype(_BF16)


def kernel(sV, sFacesOneRingIdx, sW, sJ, rV, rFacesOneRingIdx, rW, rJ,
           skeleton_idx, params):
    B, N, K = sFacesOneRingIdx.shape
    J, KJ = skeleton_idx.shape[1], skeleton_idx.shape[2]
    BB = 2 * B
    R = BB * N

    V = jnp.concatenate([sV, rV], 0).reshape(R, 3)
    idx = jnp.concatenate([sFacesOneRingIdx, rFacesOneRingIdx], 0).astype(jnp.int32)
    idxflat = (idx + (jnp.arange(BB, dtype=jnp.int32) * N)[:, None, None]).reshape(-1)

    g = params['geo']
    X0 = jnp.pad(V, ((0, 0), (0, 13)))
    X1 = _geo_layer(X0, idxflat, g['W1'], g['b1'], 3, R, K)
    X2 = _geo_layer(X1, idxflat, g['W2'], g['b2'], X1.shape[1], R, K)
    X3 = _geo_layer(X2, idxflat, g['W3'], g['b3'], X2.shape[1], R, K)

    F = jnp.concatenate([X1, X2, X3], axis=1).reshape(BB, N, -1)

    def a_stage(Wm, Jf, Fb):
        A = jnp.matmul(Wm, Fb)
        A = A / (jnp.sum(Wm, axis=-1)[..., None] + 1e-05)
        return jnp.transpose(jnp.concatenate([A, Jf], axis=-1), (0, 2, 1))

    catT = jnp.concatenate([a_stage(sW, sJ, F[:B]), a_stage(rW, rJ, F[B:])], 0)

    def sk_layer(x, idxj, W, b):
        xt = jnp.transpose(x, (0, 2, 1))
        Bn = xt.shape[0]
        nb = xt[jnp.arange(Bn)[:, None, None], idxj]
        center = xt[:, :, None, :]
        edge = jnp.concatenate([nb - center, jnp.broadcast_to(center, nb.shape)], axis=-1)
        h = _lr(edge @ W + b)
        return jnp.transpose(jnp.max(h, axis=2), (0, 2, 1))

    def sk_triple(x, idxj, p):
        x1 = sk_layer(x, idxj, p['W1'], p['b1'])
        x2 = sk_layer(x1, idxj, p['W2'], p['b2'])
        x3 = sk_layer(x2, idxj, p['W3'], p['b3'])
        return jnp.concatenate([x1, x2, x3], axis=1)

    def conv1d(x, W, b):
        return jnp.einsum('bcn,cd->bdn', x, W) + b[None, :, None]

    sF = sk_triple(catT[:B], skeleton_idx, params['skc'])
    rF = sk_triple(catT[B:], skeleton_idx, params['skc'])

    jp = params['joint']
    x = jnp.concatenate([sF, rF], axis=1)
    x = _lr(conv1d(x, jp['W1'], jp['b1']))
    x = _lr(conv1d(x, jp['W2'], jp['b2']))
    x = conv1d(x, jp['W3'], jp['b3'])
    for blk in params['res']:
        f = sk_triple(x, skeleton_idx, blk['sk'])
        f = _lr(conv1d(f, blk['W1'], blk['b1']))
        f = _lr(conv1d(f, blk['W2'], blk['b2']))
        f = conv1d(f, blk['W3'], blk['b3'])
        x = x + f
    lp = params['last']
    f = sk_triple(x, skeleton_idx, lp['sk'])
    f = _lr(conv1d(f, lp['W1'], lp['b1']))
    f = _lr(conv1d(f, lp['W2'], lp['b2']))
    f = conv1d(f, lp['W3'], lp['b3'])
    return jnp.transpose(f, (0, 2, 1))


# speedup vs baseline: 1.1559x; 1.1559x over previous
import functools

import jax
import jax.numpy as jnp
from jax import lax
from jax.experimental import pallas as pl
from jax.experimental.pallas import tpu as pltpu
from jax.experimental.pallas import tpu_sc as plsc

_F32 = jnp.float32
_BF16 = jnp.bfloat16
_NC = 2
_NS = 16
_HI = jax.lax.Precision.HIGHEST


def _lr(x):
    return jnp.where(x >= 0, x, 0.2 * x)


def _dot(a, b):
    return jnp.dot(a.astype(_BF16) if a.dtype == _F32 else a,
                   b.astype(_BF16) if b.dtype == _F32 else b,
                   preferred_element_type=_F32)


@functools.lru_cache(maxsize=None)
def _make_gather(R, C, K, P=8):
    NW = _NC * _NS
    rows_per_w = R // NW
    n_chunks = rows_per_w // P
    mesh = plsc.VectorSubcoreMesh(core_axis_name="c", subcore_axis_name="s")

    @functools.partial(
        pl.kernel,
        mesh=mesh,
        out_type=jax.ShapeDtypeStruct((R * K, C), _F32),
        scratch_types=[
            pltpu.VMEM((2, P * K), jnp.int32),
            pltpu.VMEM((2 * P * K, C), _F32),
            pltpu.SemaphoreType.DMA,
            pltpu.SemaphoreType.DMA,
        ],
        compiler_params=pltpu.CompilerParams(use_tc_tiling_on_sc=False),
    )
    def gk(x_hbm, idx_hbm, out_hbm, idx_v, rows_v, sem0, sem1):
        wid = lax.axis_index("s") * _NC + lax.axis_index("c")
        base = wid * rows_per_w
        sems = (sem0, sem1)

        def start(g, buf):
            e0 = (base + g * P) * K
            pltpu.sync_copy(idx_hbm.at[pl.ds(e0, P * K)], idx_v.at[buf])
            return pltpu.async_copy(x_hbm.at[idx_v.at[buf]],
                                    rows_v.at[pl.ds(buf * P * K, P * K)],
                                    sems[buf])

        cps = [start(0, 0)]
        for g in range(n_chunks):
            if g + 1 < n_chunks:
                cps.append(start(g + 1, (g + 1) % 2))
            cps[g].wait()
            e0 = (base + g * P) * K
            pltpu.sync_copy(rows_v.at[pl.ds((g % 2) * P * K, P * K)],
                            out_hbm.at[pl.ds(e0, P * K)])

    return gk


def _edge_body(K, TR, cin, x_ref, g_ref, w_ref, b_ref, o_ref):
    x = x_ref[:, :cin]
    g = g_ref[...].reshape(TR, K, -1)[:, :, :cin]
    d16 = (g - x[:, None, :]).astype(_BF16)
    c16 = jnp.broadcast_to(x.astype(_BF16)[:, None, :], (TR, K, cin))
    e = jnp.concatenate([d16, c16], axis=2).reshape(TR * K, 2 * cin)
    h = _dot(e, w_ref[...]).reshape(TR, K, -1) + b_ref[...][None]
    o_ref[...] = jnp.max(_lr(h), axis=1)


def _edge_mm_max(X, G, W16, b2d, K, cin, TR=256):
    R, Cinp = X.shape
    Cout = W16.shape[1]
    grid = R // TR
    return pl.pallas_call(
        functools.partial(_edge_body, K, TR, cin),
        grid=(grid,),
        in_specs=[
            pl.BlockSpec((TR, Cinp), lambda i: (i, 0)),
            pl.BlockSpec((TR * K, Cinp), lambda i: (i, 0)),
            pl.BlockSpec((2 * cin, Cout), lambda i: (0, 0)),
            pl.BlockSpec((1, Cout), lambda i: (0, 0)),
        ],
        out_specs=pl.BlockSpec((TR, Cout), lambda i: (i, 0)),
        out_shape=jax.ShapeDtypeStruct((R, Cout), _F32),
    )(X, G, W16, b2d)


def _geo_layer(Xp, idxflat, W, b, cin, R, K):
    cinp = Xp.shape[1]
    w16 = W.astype(_BF16)
    G = _make_gather(R, cinp, K)(Xp, idxflat)
    return _edge_mm_max(Xp, G, w16, b[None, :], K, cin)


def _a_body(c1, x1_ref, x2_ref, x3_ref, wm_ref, rs_ref, jf_ref, o_ref):
    w = wm_ref[0]
    a = jnp.concatenate(
        [
            _dot(w, x1_ref[:, :c1]),
            _dot(w, x2_ref[...]),
            _dot(w, x3_ref[...]),
        ],
        axis=1,
    )
    o_ref[0] = jnp.concatenate([a / rs_ref[0], jf_ref[0]], axis=1)


def _a_call(X1, X2, X3, Wm, Rs, Jf, C1):
    BB, J, N = Wm.shape
    C1p, C2, C3 = X1.shape[1], X2.shape[1], X3.shape[1]
    CJ = Jf.shape[2]
    Cout = C1 + C2 + C3 + CJ
    return pl.pallas_call(
        functools.partial(_a_body, C1),
        grid=(BB,),
        in_specs=[
            pl.BlockSpec((N, C1p), lambda i: (i, 0)),
            pl.BlockSpec((N, C2), lambda i: (i, 0)),
            pl.BlockSpec((N, C3), lambda i: (i, 0)),
            pl.BlockSpec((1, J, N), lambda i: (i, 0, 0)),
            pl.BlockSpec((1, J, 1), lambda i: (i, 0, 0)),
            pl.BlockSpec((1, J, CJ), lambda i: (i, 0, 0)),
        ],
        out_specs=pl.BlockSpec((1, J, Cout), lambda i: (i, 0, 0)),
        out_shape=jax.ShapeDtypeStruct((BB, J, Cout), _F32),
    )(X1, X2, X3, Wm, Rs, Jf)


def _onehots(ski, J, KJ):
    R2 = ski.shape[0]
    col = lax.broadcasted_iota(jnp.int32, (R2, R2), 1)
    row = lax.broadcasted_iota(jnp.int32, (R2, R2), 0)
    blk = (col // J) == (row // J)
    colm = col % J
    return [jnp.where(blk & (colm == ski[:, k:k + 1]), 1.0, 0.0).astype(_F32)
            for k in range(KJ)]


def _sk_layer(x, ohs, w16, b):
    x16 = x.astype(_BF16)
    m = None
    for oh in ohs:
        nb = jnp.dot(oh, x, preferred_element_type=_F32, precision=_HI)
        d16 = (nb - x).astype(_BF16)
        e = jnp.concatenate([d16, x16], axis=1)
        h = _lr(_dot(e, w16) + b)
        m = h if m is None else jnp.maximum(m, h)
    return m


def _sk_triple(x, ohs, w):
    x1 = _sk_layer(x, ohs, w[0][0], w[0][1])
    x2 = _sk_layer(x1, ohs, w[1][0], w[1][1])
    x3 = _sk_layer(x2, ohs, w[2][0], w[2][1])
    return jnp.concatenate([x1, x2, x3], axis=1)


def _skc_body(J, KJ, cat_ref, ski_ref, w1, b1, w2, b2, w3, b3, o_ref):
    ohs = _onehots(ski_ref[...], J, KJ)
    w = ((w1[...], b1[...]), (w2[...], b2[...]), (w3[...], b3[...]))
    o_ref[...] = _sk_triple(cat_ref[...], ohs, w)


def _skc_call(catrows, ski_rows, wlist, J, KJ, Cout):
    R2 = catrows.shape[0]
    args = [catrows, ski_rows]
    for (w16, b2d) in wlist:
        args += [w16, b2d]
    return pl.pallas_call(
        functools.partial(_skc_body, J, KJ),
        out_shape=jax.ShapeDtypeStruct((R2, Cout), _F32),
    )(*args)


def _joint_body(sf_ref, rf_ref, w1, b1, w2, b2, w3, b3, o_ref):
    x = jnp.concatenate([sf_ref[...], rf_ref[...]], axis=1)
    x = _lr(_dot(x, w1[...]) + b1[...])
    x = _lr(_dot(x, w2[...]) + b2[...])
    o_ref[...] = _dot(x, w3[...]) + b3[...]


def _joint_call(sF, rF, w1, b1, w2, b2, w3, b3):
    R2 = sF.shape[0]
    Cout = w3.shape[1]
    return pl.pallas_call(
        _joint_body,
        out_shape=jax.ShapeDtypeStruct((R2, Cout), _F32),
    )(sF, rF, w1, b1, w2, b2, w3, b3)


def _res_body(J, KJ, x_ref, ski_ref,
              sw1, sb1, sw2, sb2, sw3, sb3,
              w1, b1, w2, b2, w3, b3, o_ref):
    i = pl.program_id(0)
    x = jnp.where(i == 0, x_ref[...], o_ref[...])
    ohs = _onehots(ski_ref[...], J, KJ)
    w = ((sw1[0], sb1[0]), (sw2[0], sb2[0]), (sw3[0], sb3[0]))
    f = _sk_triple(x, ohs, w)
    f = _lr(_dot(f, w1[0]) + b1[0])
    f = _lr(_dot(f, w2[0]) + b2[0])
    f = _dot(f, w3[0]) + b3[0]
    o_ref[...] = x + f


def _res_call(x, ski_rows, stacked, J, KJ):
    R2, C = x.shape
    nblk = stacked[0].shape[0]
    full = lambda a: pl.BlockSpec(a.shape, lambda i: tuple(0 for _ in a.shape))
    perblk = lambda a: pl.BlockSpec((1,) + a.shape[1:],
                                    lambda i: (i,) + tuple(0 for _ in a.shape[1:]))
    return pl.pallas_call(
        functools.partial(_res_body, J, KJ),
        grid=(nblk,),
        in_specs=[full(x), full(ski_rows)] + [perblk(a) for a in stacked],
        out_specs=pl.BlockSpec((R2, C), lambda i: (0, 0)),
        out_shape=jax.ShapeDtypeStruct((R2, C), _F32),
    )(x, ski_rows, *stacked)


def _last_body(J, KJ, x_ref, ski_ref,
               sw1, sb1, sw2, sb2, sw3, sb3,
               w1, b1, w2, b2, w3, b3, o_ref):
    ohs = _onehots(ski_ref[...], J, KJ)
    w = ((sw1[...], sb1[...]), (sw2[...], sb2[...]), (sw3[...], sb3[...]))
    f = _sk_triple(x_ref[...], ohs, w)
    f = _lr(_dot(f, w1[...]) + b1[...])
    f = _lr(_dot(f, w2[...]) + b2[...])
    o_ref[...] = _dot(f, w3[...]) + b3[...]


def _last_call(x, ski_rows, flat, Cout, J, KJ):
    R2 = x.shape[0]
    args = [x, ski_rows] + flat
    return pl.pallas_call(
        functools.partial(_last_body, J, KJ),
        out_shape=jax.ShapeDtypeStruct((R2, Cout), _F32),
    )(*args)


def _w16(W):
    return W.astype(_BF16)


def kernel(sV, sFacesOneRingIdx, sW, sJ, rV, rFacesOneRingIdx, rW, rJ,
           skeleton_idx, params):
    B, N, K = sFacesOneRingIdx.shape
    J, KJ = skeleton_idx.shape[1], skeleton_idx.shape[2]
    BB = 2 * B
    R = BB * N

    V = jnp.concatenate([sV, rV], 0).reshape(R, 3)
    idx = jnp.concatenate([sFacesOneRingIdx, rFacesOneRingIdx], 0).astype(jnp.int32)
    idxflat = (idx + (jnp.arange(BB, dtype=jnp.int32) * N)[:, None, None]).reshape(-1)

    g = params['geo']
    X0 = jnp.pad(V, ((0, 0), (0, 13)))
    X1 = _geo_layer(X0, idxflat, g['W1'], g['b1'], 3, R, K)
    X2 = _geo_layer(X1, idxflat, g['W2'], g['b2'], X1.shape[1], R, K)
    X3 = _geo_layer(X2, idxflat, g['W3'], g['b3'], X2.shape[1], R, K)

    F = jnp.concatenate([X1, X2, X3], axis=1).reshape(BB, N, -1)

    def a_stage(Wm, Jf, Fb):
        A = jnp.matmul(Wm, Fb)
        A = A / (jnp.sum(Wm, axis=-1)[..., None] + 1e-05)
        return jnp.transpose(jnp.concatenate([A, Jf], axis=-1), (0, 2, 1))

    catT = jnp.concatenate([a_stage(sW, sJ, F[:B]), a_stage(rW, rJ, F[B:])], 0)

    def sk_layer(x, idxj, W, b):
        xt = jnp.transpose(x, (0, 2, 1))
        Bn = xt.shape[0]
        nb = xt[jnp.arange(Bn)[:, None, None], idxj]
        center = xt[:, :, None, :]
        edge = jnp.concatenate([nb - center, jnp.broadcast_to(center, nb.shape)], axis=-1)
        h = _lr(edge @ W + b)
        return jnp.transpose(jnp.max(h, axis=2), (0, 2, 1))

    def sk_triple(x, idxj, p):
        x1 = sk_layer(x, idxj, p['W1'], p['b1'])
        x2 = sk_layer(x1, idxj, p['W2'], p['b2'])
        x3 = sk_layer(x2, idxj, p['W3'], p['b3'])
        return jnp.concatenate([x1, x2, x3], axis=1)

    def conv1d(x, W, b):
        return jnp.einsum('bcn,cd->bdn', x, W) + b[None, :, None]

    sF = sk_triple(catT[:B], skeleton_idx, params['skc'])
    rF = sk_triple(catT[B:], skeleton_idx, params['skc'])

    jp = params['joint']
    x = jnp.concatenate([sF, rF], axis=1)
    x = _lr(conv1d(x, jp['W1'], jp['b1']))
    x = _lr(conv1d(x, jp['W2'], jp['b2']))
    x = conv1d(x, jp['W3'], jp['b3'])
    for blk in params['res']:
        f = sk_triple(x, skeleton_idx, blk['sk'])
        f = _lr(conv1d(f, blk['W1'], blk['b1']))
        f = _lr(conv1d(f, blk['W2'], blk['b2']))
        f = conv1d(f, blk['W3'], blk['b3'])
        x = x + f
    lp = params['last']
    f = sk_triple(x, skeleton_idx, lp['sk'])
    f = _lr(conv1d(f, lp['W1'], lp['b1']))
    f = _lr(conv1d(f, lp['W2'], lp['b2']))
    f = conv1d(f, lp['W3'], lp['b3'])
    return jnp.transpose(f, (0, 2, 1))
